# baseline (speedup 1.0000x reference)
import functools
import jax
import jax.numpy as jnp
from jax import lax
from jax.experimental import pallas as pl
from jax.experimental.pallas import tpu as pltpu
from jax.experimental.pallas import tpu_sc as plsc

DAMPING = 0.1
P = 4
NBUF = 2
LANES = 16


def _sc_body(x_hbm, pos_hbm, o_hbm, pos_t, ib0, ib1,
             psem, isem0, isem1, osem0, osem1):
    B, R, C, E = x_hbm.shape
    NCH = R // P
    w = lax.axis_index("s") * 2 + lax.axis_index("c")

    pltpu.make_async_copy(pos_hbm, pos_t, psem).start()
    pltpu.make_async_copy(pos_hbm, pos_t, psem).wait()

    ibufs = (ib0, ib1)
    isems = (isem0, isem1)
    osems = (osem0, osem1)

    def in_copy(g, k):
        return pltpu.make_async_copy(
            x_hbm.at[w, pl.ds(g * P, P)], ibufs[k], isems[k])

    def out_copy(g, k):
        return pltpu.make_async_copy(
            ibufs[k], o_hbm.at[w, pl.ds(g * P, P)], osems[k])

    def compute(g, k):
        ib = ibufs[k]
        for p in range(P):
            r = g * P + p
            for eb in range(E // LANES):
                pv = pos_t[r, pl.ds(eb * LANES, LANES)] * DAMPING
                for c in range(C):
                    plsc.addupdate(ib.at[p, c, pl.ds(eb * LANES, LANES)], pv)

    in_copy(0, 0).start()
    in_copy(0, 0).wait()
    compute(0, 0)
    out_copy(0, 0).start()
    in_copy(1, 1).start()

    def mid(sstep, carry):
        j1 = 2 * sstep + 1
        in_copy(j1, 1).wait()
        compute(j1, 1)
        out_copy(j1, 1).start()
        out_copy(j1 - 1, 0).wait()
        in_copy(j1 + 1, 0).start()
        j2 = j1 + 1
        in_copy(j2, 0).wait()
        compute(j2, 0)
        out_copy(j2, 0).start()
        out_copy(j2 - 1, 1).wait()
        in_copy(j2 + 1, 1).start()
        return carry

    lax.fori_loop(0, (NCH - 2) // 2, mid, 0)

    in_copy(NCH - 1, 1).wait()
    compute(NCH - 1, 1)
    out_copy(NCH - 1, 1).start()
    out_copy(NCH - 2, 0).wait()
    out_copy(NCH - 1, 1).wait()


def kernel(x, pos_table):
    B, R, C, E = x.shape
    mesh = plsc.VectorSubcoreMesh(core_axis_name="c", subcore_axis_name="s")
    run = functools.partial(
        pl.kernel,
        mesh=mesh,
        out_type=jax.ShapeDtypeStruct(x.shape, x.dtype),
        scratch_types=[
            pltpu.VMEM((R, E), jnp.float32),
            pltpu.VMEM((P, C, E), jnp.float32),
            pltpu.VMEM((P, C, E), jnp.float32),
            pltpu.SemaphoreType.DMA,
            pltpu.SemaphoreType.DMA,
            pltpu.SemaphoreType.DMA,
            pltpu.SemaphoreType.DMA,
            pltpu.SemaphoreType.DMA,
        ],
    )(_sc_body)
    return run(x, pos_table)


# speedup vs baseline: 1.5355x; 1.5355x over previous
import jax
import jax.numpy as jnp
from jax.experimental import pallas as pl
from jax.experimental.pallas import tpu as pltpu

DAMPING = 0.1
K = 16
NQ = 2
RC = 50


def _pos_add_kernel(x_hbm, pos_vmem, o_hbm, in_buf, out_buf, posf,
                    in_sem, out_sem):
    R, C, E = posf.shape
    B = x_hbm.shape[0]
    PER = R // RC
    N = B * PER

    posf[...] = jnp.broadcast_to(
        (pos_vmem[...] * DAMPING)[:, None, :], posf.shape)

    def in_copy(i, slot):
        b, j = jax.lax.div(i, PER), jax.lax.rem(i, PER)
        return pltpu.make_async_copy(
            x_hbm.at[b, pl.ds(j * RC, RC)], in_buf.at[slot], in_sem.at[slot])

    def out_copy(i, slot):
        b, j = jax.lax.div(i, PER), jax.lax.rem(i, PER)
        return pltpu.make_async_copy(
            out_buf.at[slot], o_hbm.at[b, pl.ds(j * RC, RC)], out_sem.at[slot])

    def start_in(i, slot):
        in_copy(i, slot).start(priority=slot % NQ)

    def start_out(i, slot):
        out_copy(i, slot).start(priority=slot % NQ)

    def compute(i, slot):
        j = jax.lax.rem(i, PER) * RC
        out_buf[slot] = in_buf[slot] + posf[pl.ds(j, RC)]

    for k in range(K):
        start_in(k, k)

    for k in range(K):
        in_copy(k, k).wait()
        compute(k, k)
        start_out(k, k)
        start_in(k + K, k)

    def mid_body(s, carry):
        base = s * K
        for k in range(K):
            i = base + k
            in_copy(i, k).wait()
            out_copy(i - K, k).wait()
            compute(i, k)
            start_out(i, k)
            start_in(i + K, k)
        return carry

    jax.lax.fori_loop(1, N // K - 1, mid_body, 0)

    for k in range(K):
        i = N - K + k
        in_copy(i, k).wait()
        out_copy(i - K, k).wait()
        compute(i, k)
        start_out(i, k)

    for k in range(K):
        out_copy(N - K + k, k).wait()


def kernel(x, pos_table):
    B, R, C, E = x.shape
    return pl.pallas_call(
        _pos_add_kernel,
        in_specs=[
            pl.BlockSpec(memory_space=pl.ANY),
            pl.BlockSpec(memory_space=pltpu.VMEM),
        ],
        out_specs=pl.BlockSpec(memory_space=pl.ANY),
        out_shape=jax.ShapeDtypeStruct(x.shape, x.dtype),
        scratch_shapes=[
            pltpu.VMEM((K, RC, C, E), jnp.float32),
            pltpu.VMEM((K, RC, C, E), jnp.float32),
            pltpu.VMEM((R, C, E), jnp.float32),
            pltpu.SemaphoreType.DMA((K,)),
            pltpu.SemaphoreType.DMA((K,)),
        ],
    )(x, pos_table)
